# Initial kernel scaffold; baseline (speedup 1.0000x reference)
import jax
import jax.numpy as jnp
from jax.experimental import pallas as pl


def kernel(logits):
    raise NotImplementedError("write your pallas kernel here")


# speedup vs baseline: 1.0758x; 1.0758x over previous
import jax
import jax.numpy as jnp
from jax.experimental import pallas as pl
from jax.experimental.pallas import tpu as pltpu

_K = 16
_EPS = 1.1754943508222875e-38


def _subset_body(x_ref, pert_ref, khot_ref, w_ref):
    x = x_ref[...]
    r, n = x.shape
    m = jnp.max(x, axis=-1, keepdims=True)
    w_ref[...] = jnp.exp(x - m)
    khot_ref[...] = jnp.zeros((r, n), jnp.float32)
    eps = jnp.float32(_EPS)
    for _ in range(_K):
        w = w_ref[...]
        s = jnp.sum(w, axis=-1, keepdims=True)
        p = w * (1.0 / s)
        khot_ref[...] += p
        w_ref[...] = w * jnp.maximum(1.0 - p, eps)

    w_ref[...] = khot_ref[...]
    pert_ref[...] = jnp.zeros((r, n), jnp.float32)
    idx = jax.lax.broadcasted_iota(jnp.int32, (r, n), 1)
    neg_inf = jnp.float32(-jnp.inf)
    for _ in range(_K):
        vals = w_ref[...]
        mx = jnp.max(vals, axis=-1, keepdims=True)
        cand = jnp.where(vals == mx, idx, jnp.int32(n))
        first = jnp.min(cand, axis=-1, keepdims=True)
        sel = idx == first
        pert_ref[...] += sel.astype(jnp.float32)
        w_ref[...] = jnp.where(sel, neg_inf, vals)

    khot = khot_ref[...]
    pert_ref[...] = (pert_ref[...] - khot) + khot


def kernel(logits):
    b, n = logits.shape
    rows = 8
    out_shape = jax.ShapeDtypeStruct((b, n), jnp.float32)
    pert, khot = pl.pallas_call(
        _subset_body,
        grid=(b // rows,),
        in_specs=[pl.BlockSpec((rows, n), lambda i: (i, 0))],
        out_specs=[pl.BlockSpec((rows, n), lambda i: (i, 0))] * 2,
        out_shape=[out_shape, out_shape],
        scratch_shapes=[pltpu.VMEM((rows, n), jnp.float32)],
    )(logits)
    return pert, khot
